# baseline (device time: 26353 ns/iter reference)
import jax
import jax.numpy as jnp
from jax import lax
from jax.experimental import pallas as pl
from jax.experimental.pallas import tpu as pltpu

C = 16


def kernel(x, pi):
    _, m, n = x.shape
    half = m // 2
    r = half // C

    def body(x_ref, pi_ref, out_ref, xstage, sendx, xrecv, yrecv,
             lsem, osemx, osemy, sx, rx, sy, ry):
        my_x = lax.axis_index("x")
        my_y = lax.axis_index("y")
        dest_x = pi_ref[my_x]

        @pl.when(dest_x == my_x)
        def _identity():
            for h in range(2):
                ld = pltpu.make_async_copy(
                    x_ref.at[0, pl.ds(h * half, half), :],
                    xstage, lsem.at[0])
                ld.start()
                ld.wait()
                xrecv[...] = xstage[...].astype(jnp.bfloat16)
                st = pltpu.make_async_copy(
                    xrecv, out_ref.at[0, pl.ds(h * half, half), :],
                    osemx.at[0])
                st.start()
                st.wait()

        @pl.when(dest_x != my_x)
        def _swap():
            half_off = my_y * half

            loads = []
            for k in range(C):
                lo = k * r
                ld = pltpu.make_async_copy(
                    x_ref.at[0, pl.ds(half_off + lo, r), :],
                    xstage.at[pl.ds(lo, r), :],
                    lsem.at[k])
                ld.start()
                loads.append(ld)

            barrier_sem = pltpu.get_barrier_semaphore()
            pl.semaphore_signal(
                barrier_sem, inc=1, device_id=(dest_x, my_y),
                device_id_type=pl.DeviceIdType.MESH)
            pl.semaphore_signal(
                barrier_sem, inc=1, device_id=(my_x, 1 - my_y),
                device_id_type=pl.DeviceIdType.MESH)
            pl.semaphore_wait(barrier_sem, 2)

            x_rdmas = []
            for k in range(C):
                lo = k * r
                loads[k].wait()
                sendx[pl.ds(lo, r), :] = xstage[
                    pl.ds(lo, r), :].astype(jnp.bfloat16)
                rdma = pltpu.make_async_remote_copy(
                    src_ref=sendx.at[pl.ds(lo, r), :],
                    dst_ref=xrecv.at[pl.ds(lo, r), :],
                    send_sem=sx.at[k],
                    recv_sem=rx.at[k],
                    device_id=(dest_x, my_y),
                    device_id_type=pl.DeviceIdType.MESH,
                )
                rdma.start()
                x_rdmas.append(rdma)

            y_rdmas, outx = [], []
            for k in range(C):
                lo = k * r
                x_rdmas[k].wait_recv()
                fwd = pltpu.make_async_remote_copy(
                    src_ref=xrecv.at[pl.ds(lo, r), :],
                    dst_ref=yrecv.at[pl.ds(lo, r), :],
                    send_sem=sy.at[k],
                    recv_sem=ry.at[k],
                    device_id=(my_x, 1 - my_y),
                    device_id_type=pl.DeviceIdType.MESH,
                )
                fwd.start()
                y_rdmas.append(fwd)
                st = pltpu.make_async_copy(
                    xrecv.at[pl.ds(lo, r), :],
                    out_ref.at[0, pl.ds(half_off + lo, r), :],
                    osemx.at[k])
                st.start()
                outx.append(st)

            other_off = (1 - my_y) * half
            outy = []
            for k in range(C):
                lo = k * r
                y_rdmas[k].wait_recv()
                st = pltpu.make_async_copy(
                    yrecv.at[pl.ds(lo, r), :],
                    out_ref.at[0, pl.ds(other_off + lo, r), :],
                    osemy.at[k])
                st.start()
                outy.append(st)

            for k in range(C):
                x_rdmas[k].wait_send()
                y_rdmas[k].wait_send()
                outx[k].wait()
                outy[k].wait()

    return pl.pallas_call(
        body,
        out_shape=jax.ShapeDtypeStruct((1, m, n), jnp.bfloat16),
        in_specs=[
            pl.BlockSpec(memory_space=pl.ANY),
            pl.BlockSpec(memory_space=pltpu.SMEM),
        ],
        out_specs=pl.BlockSpec(memory_space=pl.ANY),
        scratch_shapes=[
            pltpu.VMEM((half, n), jnp.float32),
            pltpu.VMEM((half, n), jnp.bfloat16),
            pltpu.VMEM((half, n), jnp.bfloat16),
            pltpu.VMEM((half, n), jnp.bfloat16),
            pltpu.SemaphoreType.DMA((C,)),
            pltpu.SemaphoreType.DMA((C,)),
            pltpu.SemaphoreType.DMA((C,)),
            pltpu.SemaphoreType.DMA((C,)),
            pltpu.SemaphoreType.DMA((C,)),
            pltpu.SemaphoreType.DMA((C,)),
            pltpu.SemaphoreType.DMA((C,)),
        ],
        compiler_params=pltpu.CompilerParams(collective_id=0),
    )(x, pi)


# device time: 23095 ns/iter; 1.1411x vs baseline; 1.1411x over previous
import jax
import jax.numpy as jnp
from jax import lax
from jax.experimental import pallas as pl
from jax.experimental.pallas import tpu as pltpu

C = 16


def kernel(x, pi):
    _, m, n = x.shape
    half = m // 2
    r = half // C

    def body(x_ref, pi_ref, out_ref, sendx, xrecv, yrecv,
             sx, rx, sy, ry):
        my_x = lax.axis_index("x")
        my_y = lax.axis_index("y")
        dest_x = pi_ref[my_x]

        barrier_sem = pltpu.get_barrier_semaphore()
        pl.semaphore_signal(
            barrier_sem, inc=1, device_id=(dest_x, my_y),
            device_id_type=pl.DeviceIdType.MESH)
        pl.semaphore_signal(
            barrier_sem, inc=1, device_id=(my_x, 1 - my_y),
            device_id_type=pl.DeviceIdType.MESH)
        pl.semaphore_wait(barrier_sem, 2)

        x_rdmas = []
        for k in range(C):
            lo = k * r
            sendx[pl.ds(lo, r), :] = x_ref[
                0, pl.ds(lo, r), :].astype(jnp.bfloat16)
            rdma = pltpu.make_async_remote_copy(
                src_ref=sendx.at[pl.ds(lo, r), :],
                dst_ref=xrecv.at[pl.ds(lo, r), :],
                send_sem=sx.at[k],
                recv_sem=rx.at[k],
                device_id=(dest_x, my_y),
                device_id_type=pl.DeviceIdType.MESH,
            )
            rdma.start()
            x_rdmas.append(rdma)

        half_off = my_y * half
        y_rdmas = []
        for k in range(C):
            lo = k * r
            x_rdmas[k].wait_recv()
            fwd = pltpu.make_async_remote_copy(
                src_ref=xrecv.at[pl.ds(lo, r), :],
                dst_ref=yrecv.at[pl.ds(lo, r), :],
                send_sem=sy.at[k],
                recv_sem=ry.at[k],
                device_id=(my_x, 1 - my_y),
                device_id_type=pl.DeviceIdType.MESH,
            )
            fwd.start()
            y_rdmas.append(fwd)
            out_ref[0, pl.ds(half_off + lo, r), :] = xrecv[pl.ds(lo, r), :]

        other_off = (1 - my_y) * half
        for k in range(C):
            lo = k * r
            y_rdmas[k].wait_recv()
            out_ref[0, pl.ds(other_off + lo, r), :] = yrecv[pl.ds(lo, r), :]

        for k in range(C):
            x_rdmas[k].wait_send()
            y_rdmas[k].wait_send()

    return pl.pallas_call(
        body,
        out_shape=jax.ShapeDtypeStruct((1, m, n), jnp.bfloat16),
        grid=(1,),
        in_specs=[
            pl.BlockSpec(
                (1, half, n), lambda i: (0, lax.axis_index("y"), 0)),
            pl.BlockSpec(memory_space=pltpu.SMEM),
        ],
        out_specs=pl.BlockSpec((1, m, n), lambda i: (0, 0, 0)),
        scratch_shapes=[
            pltpu.VMEM((half, n), jnp.bfloat16),
            pltpu.VMEM((half, n), jnp.bfloat16),
            pltpu.VMEM((half, n), jnp.bfloat16),
            pltpu.SemaphoreType.DMA((C,)),
            pltpu.SemaphoreType.DMA((C,)),
            pltpu.SemaphoreType.DMA((C,)),
            pltpu.SemaphoreType.DMA((C,)),
        ],
        compiler_params=pltpu.CompilerParams(collective_id=0),
    )(x, pi)


# device time: 22389 ns/iter; 1.1771x vs baseline; 1.0315x over previous
import jax
import jax.numpy as jnp
from jax import lax
from jax.experimental import pallas as pl
from jax.experimental.pallas import tpu as pltpu

C = 16


def kernel(x, pi):
    _, m, n = x.shape
    half = m // 2
    r = half // C

    x = pltpu.with_memory_space_constraint(x, pltpu.MemorySpace.HBM)
    pi = pltpu.with_memory_space_constraint(pi, pltpu.MemorySpace.SMEM)

    def body(x_ref, pi_ref, out_ref, sendx, xrecv, yrecv,
             sx, rx, sy, ry):
        my_x = lax.axis_index("x")
        my_y = lax.axis_index("y")
        dest_x = pi_ref[my_x]

        barrier_sem = pltpu.get_barrier_semaphore()
        pl.semaphore_signal(
            barrier_sem, inc=1, device_id=(dest_x, my_y),
            device_id_type=pl.DeviceIdType.MESH)
        pl.semaphore_signal(
            barrier_sem, inc=1, device_id=(my_x, 1 - my_y),
            device_id_type=pl.DeviceIdType.MESH)
        pl.semaphore_wait(barrier_sem, 2)

        x_rdmas = []
        for k in range(C):
            lo = k * r
            sendx[pl.ds(lo, r), :] = x_ref[
                0, pl.ds(lo, r), :].astype(jnp.bfloat16)
            rdma = pltpu.make_async_remote_copy(
                src_ref=sendx.at[pl.ds(lo, r), :],
                dst_ref=xrecv.at[pl.ds(lo, r), :],
                send_sem=sx.at[k],
                recv_sem=rx.at[k],
                device_id=(dest_x, my_y),
                device_id_type=pl.DeviceIdType.MESH,
            )
            rdma.start()
            x_rdmas.append(rdma)

        half_off = my_y * half
        y_rdmas = []
        for k in range(C):
            lo = k * r
            x_rdmas[k].wait_recv()
            fwd = pltpu.make_async_remote_copy(
                src_ref=xrecv.at[pl.ds(lo, r), :],
                dst_ref=yrecv.at[pl.ds(lo, r), :],
                send_sem=sy.at[k],
                recv_sem=ry.at[k],
                device_id=(my_x, 1 - my_y),
                device_id_type=pl.DeviceIdType.MESH,
            )
            fwd.start()
            y_rdmas.append(fwd)
            out_ref[0, pl.ds(half_off + lo, r), :] = xrecv[pl.ds(lo, r), :]

        other_off = (1 - my_y) * half
        for k in range(C):
            lo = k * r
            y_rdmas[k].wait_recv()
            out_ref[0, pl.ds(other_off + lo, r), :] = yrecv[pl.ds(lo, r), :]

        for k in range(C):
            x_rdmas[k].wait_send()
            y_rdmas[k].wait_send()

    return pl.pallas_call(
        body,
        out_shape=jax.ShapeDtypeStruct((1, m, n), jnp.bfloat16),
        grid=(1,),
        in_specs=[
            pl.BlockSpec(
                (1, half, n), lambda i: (0, lax.axis_index("y"), 0)),
            pl.BlockSpec(memory_space=pltpu.SMEM),
        ],
        out_specs=pl.BlockSpec((1, m, n), lambda i: (0, 0, 0)),
        scratch_shapes=[
            pltpu.VMEM((half, n), jnp.bfloat16),
            pltpu.VMEM((half, n), jnp.bfloat16),
            pltpu.VMEM((half, n), jnp.bfloat16),
            pltpu.SemaphoreType.DMA((C,)),
            pltpu.SemaphoreType.DMA((C,)),
            pltpu.SemaphoreType.DMA((C,)),
            pltpu.SemaphoreType.DMA((C,)),
        ],
        compiler_params=pltpu.CompilerParams(collective_id=0),
    )(x, pi)
